# Initial kernel scaffold; baseline (speedup 1.0000x reference)
import jax
import jax.numpy as jnp
from jax.experimental import pallas as pl


def kernel(node_feats, edge_index, edge_feats, glob_feats, batch, mW1, mb1, aW1, ab1):
    raise NotImplementedError("write your pallas kernel here")


# speedup vs baseline: 3.9262x; 3.9262x over previous
import functools

import jax
import jax.numpy as jnp
from jax import lax
from jax.experimental import pallas as pl
from jax.experimental.pallas import tpu as pltpu
from jax.experimental.pallas import tpu_sc as plsc

NC, NS, LANES = 2, 16, 16
MSGW = 64
AW = 128
CHUNK = 80


def _mm_bias_kernel(x_ref, w_ref, b_ref, o_ref):
    o_ref[...] = (
        jnp.dot(x_ref[...], w_ref[...], preferred_element_type=jnp.float32)
        + b_ref[...]
    )


def _final_kernel(node_ref, sums_ref, batch_ref, glob_ref, a1_ref, a2_ref,
                  a3_ref, ab_ref, o_ref):
    s = sums_ref[0] + sums_ref[1]
    cnt = s[:, MSGW:MSGW + 1]
    mean = s[:, :MSGW] / jnp.maximum(cnt, 1.0)
    g_tab = jnp.dot(glob_ref[...], a3_ref[...],
                    preferred_element_type=jnp.float32)
    b = batch_ref[...]
    nb = g_tab.shape[0]
    onehot = (b == lax.broadcasted_iota(jnp.int32, (b.shape[0], nb), 1))
    acc = (
        jnp.dot(node_ref[...], a1_ref[...], preferred_element_type=jnp.float32)
        + jnp.dot(mean, a2_ref[...], preferred_element_type=jnp.float32)
        + jnp.dot(onehot.astype(jnp.float32), g_tab,
                  preferred_element_type=jnp.float32)
        + ab_ref[...]
    )
    o_ref[...] = jnp.maximum(acc, 0.0)


@functools.lru_cache(maxsize=None)
def _build_sc_edges(N, E):
    W = NC * NS
    epw = E // W
    nch = epw // CHUNK
    rpt = (-(-N // NS) + 7) // 8 * 8
    npad = NS * rpt

    mesh = plsc.VectorSubcoreMesh(core_axis_name="c", subcore_axis_name="s")

    @functools.partial(
        pl.kernel,
        out_type=jax.ShapeDtypeStruct((NC, npad, AW), jnp.float32),
        mesh=mesh,
        scratch_types=[
            pltpu.VMEM_SHARED((npad, AW), jnp.float32),
            pltpu.VMEM((epw,), jnp.int32),
            pltpu.VMEM((epw,), jnp.int32),
            pltpu.VMEM((CHUNK,), jnp.int32),
            pltpu.VMEM((CHUNK, AW), jnp.float32),
            pltpu.VMEM((CHUNK, MSGW), jnp.float32),
            pltpu.SemaphoreType.DMA,
            pltpu.SemaphoreType.DMA,
        ],
    )
    def sc_edges(p_hbm, q_hbm, src_hbm, dst_hbm, zero_hbm, out_hbm,
                 accum, sidx, didx, dbuf, msg, qrows, gsem, qsem):
        cid = lax.axis_index("c")
        sid = lax.axis_index("s")
        w = cid * NS + sid

        pltpu.sync_copy(src_hbm.at[pl.ds(w * epw, epw)], sidx)
        pltpu.sync_copy(dst_hbm.at[pl.ds(w * epw, epw)], didx)
        pltpu.sync_copy(zero_hbm.at[pl.ds(sid * rpt, rpt)],
                        accum.at[pl.ds(sid * rpt, rpt)])

        plsc.subcore_barrier()

        ebase = w * epw

        def chunk_body(jc, carry):
            g = pltpu.async_copy(
                p_hbm.at[sidx.at[pl.ds(jc * CHUNK, CHUNK)]], msg, gsem)
            qd = pltpu.async_copy(q_hbm.at[pl.ds(ebase + jc * CHUNK, CHUNK)],
                                  qrows, qsem)
            @plsc.parallel_loop(0, CHUNK // LANES)
            def _didx_copy(jj):
                dbuf[pl.ds(jj * LANES, LANES)] = (
                    didx[pl.ds(jc * CHUNK + jj * LANES, LANES)])
            g.wait()
            qd.wait()

            @plsc.parallel_loop(0, CHUNK, unroll=8)
            def _relu(i):
                for jj in range(MSGW // LANES):
                    sl = pl.ds(jj * LANES, LANES)
                    msg[i, sl] = jnp.maximum(msg[i, sl] + qrows[i, sl], 0.0)

            pltpu.sync_copy(msg, accum.at[dbuf], add=True)
            return carry

        lax.fori_loop(0, nch, chunk_body, 0)

        plsc.subcore_barrier()

        pltpu.sync_copy(accum.at[pl.ds(sid * rpt, rpt)],
                        out_hbm.at[cid, pl.ds(sid * rpt, rpt)])

    return sc_edges


def kernel(node_feats, edge_index, edge_feats, glob_feats, batch, mW1, mb1,
           aW1, ab1):
    N, D = node_feats.shape
    E, EIN = edge_feats.shape
    B = glob_feats.shape[0]
    f32 = jnp.float32

    mW1a = mW1[:D]
    mW1b = mW1[D:]
    A1 = aW1[:D]
    A2 = aW1[D:D + MSGW]
    A3 = aW1[D + MSGW:]
    AOUT = aW1.shape[1]

    wpad = jnp.concatenate([mW1a, jnp.zeros((D, AW - MSGW), f32)], axis=1)
    bias_row = jnp.zeros((1, AW), f32).at[0, MSGW].set(1.0)
    rb = 2000
    P2 = pl.pallas_call(
        _mm_bias_kernel,
        grid=(N // rb,),
        in_specs=[
            pl.BlockSpec((rb, D), lambda i: (i, 0)),
            pl.BlockSpec((D, AW), lambda i: (0, 0)),
            pl.BlockSpec((1, AW), lambda i: (0, 0)),
        ],
        out_specs=pl.BlockSpec((rb, AW), lambda i: (i, 0)),
        out_shape=jax.ShapeDtypeStruct((N, AW), f32),
    )(node_feats, wpad, bias_row)

    eb = 4000
    Q = pl.pallas_call(
        _mm_bias_kernel,
        grid=(E // eb,),
        in_specs=[
            pl.BlockSpec((eb, EIN), lambda i: (i, 0)),
            pl.BlockSpec((EIN, MSGW), lambda i: (0, 0)),
            pl.BlockSpec((1, MSGW), lambda i: (0, 0)),
        ],
        out_specs=pl.BlockSpec((eb, MSGW), lambda i: (i, 0)),
        out_shape=jax.ShapeDtypeStruct((E, MSGW), f32),
    )(edge_feats, mW1b, mb1.reshape(1, MSGW))

    rpt = (-(-N // NS) + 7) // 8 * 8
    npad = NS * rpt
    zeros = jnp.zeros((npad, AW), f32)
    sums2 = _build_sc_edges(N, E)(P2, Q, edge_index[0], edge_index[1], zeros)
    sums2 = sums2[:, :N]

    fb = 2000
    out = pl.pallas_call(
        _final_kernel,
        grid=(N // fb,),
        in_specs=[
            pl.BlockSpec((fb, D), lambda i: (i, 0)),
            pl.BlockSpec((NC, fb, AW), lambda i: (0, i, 0)),
            pl.BlockSpec((fb, 1), lambda i: (i, 0)),
            pl.BlockSpec((B, EIN), lambda i: (0, 0)),
            pl.BlockSpec((D, AOUT), lambda i: (0, 0)),
            pl.BlockSpec((MSGW, AOUT), lambda i: (0, 0)),
            pl.BlockSpec((EIN, AOUT), lambda i: (0, 0)),
            pl.BlockSpec((1, AOUT), lambda i: (0, 0)),
        ],
        out_specs=pl.BlockSpec((fb, AOUT), lambda i: (i, 0)),
        out_shape=jax.ShapeDtypeStruct((N, AOUT), f32),
    )(node_feats, sums2, batch.reshape(N, 1), glob_feats, A1, A2, A3,
      ab1.reshape(1, AOUT))

    return out


# speedup vs baseline: 4.6828x; 1.1927x over previous
import functools

import jax
import jax.numpy as jnp
from jax import lax
from jax.experimental import pallas as pl
from jax.experimental.pallas import tpu as pltpu
from jax.experimental.pallas import tpu_sc as plsc

NC, NS, LANES = 2, 16, 16
MSGW = 64
AW = 128
SCH = 40


def _mm_bias_kernel(x_ref, w_ref, b_ref, o_ref):
    o_ref[...] = (
        jnp.dot(x_ref[...], w_ref[...], preferred_element_type=jnp.float32)
        + b_ref[...]
    )


def _qpack_kernel(xa_ref, xb_ref, w_ref, b_ref, o_ref):
    qa = jnp.dot(xa_ref[...], w_ref[...], preferred_element_type=jnp.float32)
    qb = jnp.dot(xb_ref[...], w_ref[...], preferred_element_type=jnp.float32)
    o_ref[...] = jnp.concatenate([qa + b_ref[...], qb + b_ref[...]], axis=1)


def _final_kernel(node_ref, sums_ref, batch_ref, glob_ref, a1_ref, a2_ref,
                  a3_ref, ab_ref, o_ref):
    s = sums_ref[0] + sums_ref[1]
    cnt = s[:, MSGW:MSGW + 1]
    mean = s[:, :MSGW] / jnp.maximum(cnt, 1.0)
    g_tab = jnp.dot(glob_ref[...], a3_ref[...],
                    preferred_element_type=jnp.float32)
    b = batch_ref[...]
    nb = g_tab.shape[0]
    onehot = (b == lax.broadcasted_iota(jnp.int32, (b.shape[0], nb), 1))
    acc = (
        jnp.dot(node_ref[...], a1_ref[...], preferred_element_type=jnp.float32)
        + jnp.dot(mean, a2_ref[...], preferred_element_type=jnp.float32)
        + jnp.dot(onehot.astype(jnp.float32), g_tab,
                  preferred_element_type=jnp.float32)
        + ab_ref[...]
    )
    o_ref[...] = jnp.maximum(acc, 0.0)


@functools.lru_cache(maxsize=None)
def _build_sc_edges(N, E):
    W = NC * NS
    epw = E // W
    half = epw // 2
    nch = half // SCH
    rpt = (-(-N // NS) + 7) // 8 * 8
    npad = NS * rpt
    NB = 2

    mesh = plsc.VectorSubcoreMesh(core_axis_name="c", subcore_axis_name="s")

    @functools.partial(
        pl.kernel,
        out_type=jax.ShapeDtypeStruct((NC, npad, AW), jnp.float32),
        mesh=mesh,
        scratch_types=[
            pltpu.VMEM_SHARED((npad, AW), jnp.float32),
            pltpu.VMEM((epw,), jnp.int32),
            [pltpu.VMEM((SCH,), jnp.int32) for _ in range(NB)],
            [pltpu.VMEM((SCH,), jnp.int32) for _ in range(NB)],
            [pltpu.VMEM((SCH, AW), jnp.float32) for _ in range(NB)],
            [pltpu.VMEM((SCH, AW), jnp.float32) for _ in range(NB)],
            [pltpu.VMEM((SCH, AW), jnp.float32) for _ in range(NB)],
            [pltpu.SemaphoreType.DMA for _ in range(NB)],
            [pltpu.SemaphoreType.DMA for _ in range(NB)],
            [pltpu.SemaphoreType.DMA for _ in range(NB)],
            [pltpu.SemaphoreType.DMA for _ in range(NB)],
            [pltpu.SemaphoreType.DMA for _ in range(NB)],
            [pltpu.SemaphoreType.DMA for _ in range(NB)],
            [pltpu.SemaphoreType.DMA for _ in range(NB)],
        ],
    )
    def sc_edges(p_hbm, q_hbm, src_hbm, dst_hbm, zero_hbm, out_hbm,
                 accum, sidx, dbufa, dbufb, msga, msgb, qbuf,
                 qsem, gsa, gsb, dsa, dsb, ssa, ssb):
        cid = lax.axis_index("c")
        sid = lax.axis_index("s")
        w = cid * NS + sid
        ebase = w * epw
        qrow0 = w * half

        pltpu.sync_copy(src_hbm.at[pl.ds(ebase, epw)], sidx)
        pltpu.sync_copy(zero_hbm.at[pl.ds(sid * rpt, rpt)],
                        accum.at[pl.ds(sid * rpt, rpt)])

        plsc.subcore_barrier()

        def issue(jc, b):
            o = jc * SCH
            pltpu.async_copy(dst_hbm.at[pl.ds(ebase + o, SCH)],
                             dbufa[b], dsa[b])
            pltpu.async_copy(dst_hbm.at[pl.ds(ebase + half + o, SCH)],
                             dbufb[b], dsb[b])
            pltpu.async_copy(q_hbm.at[pl.ds(qrow0 + o, SCH)], qbuf[b], qsem[b])
            pltpu.async_copy(p_hbm.at[sidx.at[pl.ds(o, SCH)]], msga[b], gsa[b])
            pltpu.async_copy(p_hbm.at[sidx.at[pl.ds(half + o, SCH)]],
                             msgb[b], gsb[b])

        def wait_in(jc, b):
            pltpu.make_async_copy(q_hbm.at[pl.ds(0, SCH)], qbuf[b],
                                  qsem[b]).wait()
            pltpu.make_async_copy(p_hbm.at[sidx.at[pl.ds(0, SCH)]], msga[b],
                                  gsa[b]).wait()
            pltpu.make_async_copy(p_hbm.at[sidx.at[pl.ds(0, SCH)]], msgb[b],
                                  gsb[b]).wait()
            pltpu.make_async_copy(dst_hbm.at[pl.ds(0, SCH)], dbufa[b],
                                  dsa[b]).wait()
            pltpu.make_async_copy(dst_hbm.at[pl.ds(0, SCH)], dbufb[b],
                                  dsb[b]).wait()

        def process(jc, b):
            wait_in(jc, b)

            @plsc.parallel_loop(0, SCH, unroll=4)
            def _relu(i):
                for jj in range(MSGW // LANES):
                    sl = pl.ds(jj * LANES, LANES)
                    sr = pl.ds(MSGW + jj * LANES, LANES)
                    msga[b][i, sl] = jnp.maximum(
                        msga[b][i, sl] + qbuf[b][i, sl], 0.0)
                    msgb[b][i, sl] = jnp.maximum(
                        msgb[b][i, sl] + qbuf[b][i, sr], 0.0)

            pltpu.async_copy(msga[b], accum.at[dbufa[b]], ssa[b], add=True)
            pltpu.async_copy(msgb[b], accum.at[dbufb[b]], ssb[b], add=True)

        def wait_scatter(b):
            pltpu.make_async_copy(msga[b], accum.at[dbufa[b]], ssa[b]).wait()
            pltpu.make_async_copy(msgb[b], accum.at[dbufb[b]], ssb[b]).wait()

        issue(0, 0)
        issue(1, 1)

        def round_body(k, carry):
            jc0 = 2 * k

            process(jc0, 0)

            @pl.when(jc0 + 2 < nch)
            def _():
                wait_scatter(0)
                issue(jc0 + 2, 0)

            process(jc0 + 1, 1)

            @pl.when(jc0 + 3 < nch)
            def _():
                wait_scatter(1)
                issue(jc0 + 3, 1)

            return carry

        lax.fori_loop(0, nch // 2, round_body, 0)
        if nch % 2:
            process(nch - 1, 0)
        wait_scatter(0)
        wait_scatter(1)

        plsc.subcore_barrier()

        pltpu.sync_copy(accum.at[pl.ds(sid * rpt, rpt)],
                        out_hbm.at[cid, pl.ds(sid * rpt, rpt)])

    return sc_edges


def kernel(node_feats, edge_index, edge_feats, glob_feats, batch, mW1, mb1,
           aW1, ab1):
    N, D = node_feats.shape
    E, EIN = edge_feats.shape
    B = glob_feats.shape[0]
    f32 = jnp.float32

    mW1a = mW1[:D]
    mW1b = mW1[D:]
    A1 = aW1[:D]
    A2 = aW1[D:D + MSGW]
    A3 = aW1[D + MSGW:]
    AOUT = aW1.shape[1]

    wpad = jnp.concatenate([mW1a, jnp.zeros((D, AW - MSGW), f32)], axis=1)
    bias_row = jnp.zeros((1, AW), f32).at[0, MSGW].set(1.0)
    rb = 2000
    P2 = pl.pallas_call(
        _mm_bias_kernel,
        grid=(N // rb,),
        in_specs=[
            pl.BlockSpec((rb, D), lambda i: (i, 0)),
            pl.BlockSpec((D, AW), lambda i: (0, 0)),
            pl.BlockSpec((1, AW), lambda i: (0, 0)),
        ],
        out_specs=pl.BlockSpec((rb, AW), lambda i: (i, 0)),
        out_shape=jax.ShapeDtypeStruct((N, AW), f32),
    )(node_feats, wpad, bias_row)

    W = NC * NS
    epw = E // W
    half = epw // 2
    eb = 1000
    k = half // eb

    Q = pl.pallas_call(
        _qpack_kernel,
        grid=(W, k),
        in_specs=[
            pl.BlockSpec((eb, EIN), lambda w, p: (2 * k * w + p, 0)),
            pl.BlockSpec((eb, EIN), lambda w, p: (2 * k * w + k + p, 0)),
            pl.BlockSpec((EIN, MSGW), lambda w, p: (0, 0)),
            pl.BlockSpec((1, MSGW), lambda w, p: (0, 0)),
        ],
        out_specs=pl.BlockSpec((eb, 2 * MSGW), lambda w, p: (k * w + p, 0)),
        out_shape=jax.ShapeDtypeStruct((E // 2, 2 * MSGW), f32),
    )(edge_feats, edge_feats, mW1b, mb1.reshape(1, MSGW))

    rpt = (-(-N // NS) + 7) // 8 * 8
    npad = NS * rpt
    zeros = jnp.zeros((npad, AW), f32)
    sums2 = _build_sc_edges(N, E)(P2, Q, edge_index[0], edge_index[1], zeros)
    sums2 = sums2[:, :N]

    fb = 2000
    out = pl.pallas_call(
        _final_kernel,
        grid=(N // fb,),
        in_specs=[
            pl.BlockSpec((fb, D), lambda i: (i, 0)),
            pl.BlockSpec((NC, fb, AW), lambda i: (0, i, 0)),
            pl.BlockSpec((fb, 1), lambda i: (i, 0)),
            pl.BlockSpec((B, EIN), lambda i: (0, 0)),
            pl.BlockSpec((D, AOUT), lambda i: (0, 0)),
            pl.BlockSpec((MSGW, AOUT), lambda i: (0, 0)),
            pl.BlockSpec((EIN, AOUT), lambda i: (0, 0)),
            pl.BlockSpec((1, AOUT), lambda i: (0, 0)),
        ],
        out_specs=pl.BlockSpec((fb, AOUT), lambda i: (i, 0)),
        out_shape=jax.ShapeDtypeStruct((N, AOUT), f32),
    )(node_feats, sums2, batch.reshape(N, 1), glob_feats, A1, A2, A3,
      ab1.reshape(1, AOUT))

    return out
